# baseline (device time: 105510 ns/iter reference)
import jax
import jax.numpy as jnp
from jax import lax
from jax.experimental import pallas as pl
from jax.experimental.pallas import tpu as pltpu

N_DEV = 16


def kernel(A, B):
    m, k = A.shape
    _, n = B.shape
    grp = m // 4
    half = grp // 2
    chunk = m // N_DEV
    nh = n // 4

    f32 = jnp.float32
    bf16 = jnp.bfloat16
    MESH = pl.DeviceIdType.MESH

    def body(a_ref, b_ref, out_ref, comm_r, comm_l, zred_ref, zrs_ref,
             r_send, r_recv, l_send, l_recv,
             zrs_send, zrs_recv, zag_send, zag_recv,
             ag_send, ag_recv):
        my = lax.axis_index("i")
        p = jnp.mod(my, 4)
        zpos = my // 4
        base = my - p

        def pr_id(q):
            return base + jnp.mod(q, 4)

        def col_id(zq):
            return jnp.mod(zq, 4) * 4 + p

        right = pr_id(p + 1)
        left = pr_id(p - 1)
        my_rows = pl.ds(p * grp + zpos * chunk, chunk)

        def cols(h):
            return pl.ds(h * nh, nh)

        def top_rows(g):
            return pl.ds(jnp.mod(g, 4) * grp, half)

        def bot_rows(g):
            return pl.ds(jnp.mod(g, 4) * grp + half, half)

        def partial(row_ds, h):
            return jnp.dot(a_ref[row_ds, :], b_ref[:, cols(h)],
                           preferred_element_type=f32)

        def copy(src, dst, send, recv, dev):
            return pltpu.make_async_remote_copy(
                src_ref=src, dst_ref=dst, send_sem=send, recv_sem=recv,
                device_id=(dev,), device_id_type=MESH)

        send_descs = []


        def p1_stage(h):
            comm_r[h, 0, :, :] = partial(top_rows(p + 3), h).astype(bf16)
            comm_l[h, 0, :, :] = partial(bot_rows(p + 1), h).astype(bf16)

        def p1_start(h, s):
            rr = copy(comm_r.at[h, s], comm_r.at[h, s + 1],
                      r_send.at[h * 3 + s], r_recv.at[h * 3 + s], right)
            ll = copy(comm_l.at[h, s], comm_l.at[h, s + 1],
                      l_send.at[h * 3 + s], l_recv.at[h * 3 + s], left)
            rr.start()
            ll.start()
            return rr, ll

        def p1_finish(h, s, rr, ll):
            p_r = partial(top_rows(p + 2 - s), h)
            p_l = partial(bot_rows(p + 2 + s), h)
            rr.wait()
            ll.wait()
            if s < 2:
                comm_r[h, s + 1, :, :] = (
                    comm_r[h, s + 1, :, :].astype(f32) + p_r).astype(bf16)
                comm_l[h, s + 1, :, :] = (
                    comm_l[h, s + 1, :, :].astype(f32) + p_l).astype(bf16)
            else:
                zred_ref[h, pl.ds(0, half), :] = (
                    comm_r[h, 3, :, :].astype(f32) + p_r).astype(bf16)
                zred_ref[h, pl.ds(half, half), :] = (
                    comm_l[h, 3, :, :].astype(f32) + p_l).astype(bf16)

        def p2_send(h):
            for dz in range(1, 4):
                zq = jnp.mod(zpos + dz, 4)
                d = copy(zred_ref.at[h, pl.ds(zq * chunk, chunk), :],
                         zrs_ref.at[h, pl.ds(zpos * chunk, chunk), :],
                         zrs_send.at[h * 3 + dz - 1],
                         zrs_recv.at[h * 4 + zpos], col_id(zpos + dz))
                d.start()
                send_descs.append(d)

        def p2_reduce_silu(h):
            acc = zred_ref[h, pl.ds(zpos * chunk, chunk), :].astype(f32)
            for dz in range(1, 4):
                zq = jnp.mod(zpos + dz, 4)
                slot = zrs_ref.at[h, pl.ds(zq * chunk, chunk), :]
                copy(slot, slot, zrs_send.at[0],
                     zrs_recv.at[h * 4 + zq], my).wait_recv()
                acc = acc + zrs_ref[h, pl.ds(zq * chunk, chunk), :].astype(f32)
            zs = acc / (1.0 + jnp.exp(-acc))
            out_ref[my_rows, cols(h)] = zs.astype(bf16)

        def p3_send(h):
            for dz in range(1, 4):
                d = copy(out_ref.at[my_rows, cols(h)],
                         out_ref.at[my_rows, cols(h)],
                         zag_send.at[h * 3 + dz - 1],
                         zag_recv.at[h * 4 + zpos], col_id(zpos + dz))
                d.start()
                send_descs.append(d)

        def p3_wait(h):
            for dz in range(1, 4):
                zq = jnp.mod(zpos + dz, 4)
                slot = out_ref.at[pl.ds(p * grp + zq * chunk, chunk), cols(h)]
                copy(slot, slot, zag_send.at[0],
                     zag_recv.at[h * 4 + zq], my).wait_recv()

        def p4_start(h, s):
            sr = copy(out_ref.at[top_rows(p - s), cols(h)],
                      out_ref.at[top_rows(p - s), cols(h)],
                      ag_send.at[h * 6 + s], ag_recv.at[h * 6 + s], right)
            sl = copy(out_ref.at[bot_rows(p + s), cols(h)],
                      out_ref.at[bot_rows(p + s), cols(h)],
                      ag_send.at[h * 6 + 3 + s], ag_recv.at[h * 6 + 3 + s],
                      left)
            sr.start()
            sl.start()
            send_descs.append(sr)
            send_descs.append(sl)

        def p4_finish(h, s):
            slot_r = out_ref.at[top_rows(p - s - 1), cols(h)]
            copy(slot_r, slot_r, ag_send.at[0],
                 ag_recv.at[h * 6 + s], my).wait_recv()
            slot_l = out_ref.at[bot_rows(p + s + 1), cols(h)]
            copy(slot_l, slot_l, ag_send.at[0],
                 ag_recv.at[h * 6 + 3 + s], my).wait_recv()

        partners = (left, right, pr_id(p + 2),
                    col_id(zpos + 1), col_id(zpos + 2), col_id(zpos + 3))
        barrier = pltpu.get_barrier_semaphore()
        for tgt in partners:
            pl.semaphore_signal(barrier, inc=1, device_id=(tgt,),
                                device_id_type=MESH)
        pl.semaphore_wait(barrier, 6)

        d = {}
        p1_stage(0)
        d[0] = p1_start(0, 0)
        p1_stage(1)
        d[1] = p1_start(1, 0)
        p1_finish(0, 0, *d[0])
        d[0] = p1_start(0, 1)
        p1_stage(2)
        d[2] = p1_start(2, 0)
        p1_finish(1, 0, *d[1])
        d[1] = p1_start(1, 1)
        p1_finish(0, 1, *d[0])
        d[0] = p1_start(0, 2)
        p1_stage(3)
        d[3] = p1_start(3, 0)
        p1_finish(2, 0, *d[2])
        d[2] = p1_start(2, 1)
        p1_finish(1, 1, *d[1])
        d[1] = p1_start(1, 2)
        p1_finish(0, 2, *d[0])
        p2_send(0)
        p1_finish(3, 0, *d[3])
        d[3] = p1_start(3, 1)
        p1_finish(2, 1, *d[2])
        d[2] = p1_start(2, 2)
        p2_reduce_silu(0)
        p3_send(0)
        p1_finish(1, 2, *d[1])
        p2_send(1)
        p1_finish(3, 1, *d[3])
        d[3] = p1_start(3, 2)
        p3_wait(0)
        p2_reduce_silu(1)
        p3_send(1)
        p4_start(0, 0)
        p1_finish(2, 2, *d[2])
        p2_send(2)
        p4_finish(0, 0)
        p4_start(0, 1)
        p3_wait(1)
        p1_finish(3, 2, *d[3])
        p2_send(3)
        p2_reduce_silu(2)
        p3_send(2)
        p4_start(1, 0)
        p4_finish(0, 1)
        p4_start(0, 2)
        p3_wait(2)
        p2_reduce_silu(3)
        p3_send(3)
        p4_start(2, 0)
        p4_finish(1, 0)
        p4_start(1, 1)
        p4_finish(0, 2)
        p3_wait(3)
        p4_finish(2, 0)
        p4_start(2, 1)
        p4_start(3, 0)
        p4_finish(1, 1)
        p4_start(1, 2)
        p4_finish(2, 1)
        p4_start(2, 2)
        p4_finish(3, 0)
        p4_start(3, 1)
        p4_finish(1, 2)
        p4_finish(2, 2)
        p4_finish(3, 1)
        p4_start(3, 2)
        p4_finish(3, 2)

        for d in send_descs:
            d.wait_send()

    return pl.pallas_call(
        body,
        out_shape=jax.ShapeDtypeStruct((m, n), bf16),
        in_specs=[
            pl.BlockSpec(memory_space=pltpu.VMEM),
            pl.BlockSpec(memory_space=pltpu.VMEM),
        ],
        out_specs=pl.BlockSpec(memory_space=pltpu.VMEM),
        scratch_shapes=[
            pltpu.VMEM((4, 4, half, nh), bf16),
            pltpu.VMEM((4, 4, half, nh), bf16),
            pltpu.VMEM((4, grp, nh), bf16),
            pltpu.VMEM((4, grp, nh), bf16),
            pltpu.SemaphoreType.DMA((12,)),
            pltpu.SemaphoreType.DMA((12,)),
            pltpu.SemaphoreType.DMA((12,)),
            pltpu.SemaphoreType.DMA((12,)),
            pltpu.SemaphoreType.DMA((12,)),
            pltpu.SemaphoreType.DMA((16,)),
            pltpu.SemaphoreType.DMA((12,)),
            pltpu.SemaphoreType.DMA((16,)),
            pltpu.SemaphoreType.DMA((24,)),
            pltpu.SemaphoreType.DMA((24,)),
        ],
        compiler_params=pltpu.CompilerParams(
            collective_id=0,
            vmem_limit_bytes=45 * 1024 * 1024,
        ),
    )(A, B)


# device time: 98239 ns/iter; 1.0740x vs baseline; 1.0740x over previous
import jax
import jax.numpy as jnp
from jax import lax
from jax.experimental import pallas as pl
from jax.experimental.pallas import tpu as pltpu

N_DEV = 16


def kernel(A, B):
    m, k = A.shape
    _, n = B.shape
    grp = m // 4
    half = grp // 2
    chunk = m // N_DEV
    nh = n // 4

    f32 = jnp.float32
    bf16 = jnp.bfloat16
    MESH = pl.DeviceIdType.MESH

    def body(a_ref, b_ref, out_ref, comm_r, comm_l, zred_ref, zrs_ref,
             r_send, r_recv, l_send, l_recv,
             zrs_send, zrs_recv, zag_send, zag_recv,
             ag_send, ag_recv):
        my = lax.axis_index("i")
        p = jnp.mod(my, 4)
        zpos = my // 4
        base = my - p

        def pr_id(q):
            return base + jnp.mod(q, 4)

        def col_id(zq):
            return jnp.mod(zq, 4) * 4 + p

        right = pr_id(p + 1)
        left = pr_id(p - 1)
        my_rows = pl.ds(p * grp + zpos * chunk, chunk)

        def cols(h):
            return pl.ds(h * nh, nh)

        def top_rows(g):
            return pl.ds(jnp.mod(g, 4) * grp, half)

        def bot_rows(g):
            return pl.ds(jnp.mod(g, 4) * grp + half, half)

        def partial(row_ds, h):
            return jnp.dot(a_ref[row_ds, :], b_ref[:, cols(h)],
                           preferred_element_type=f32)

        def copy(src, dst, send, recv, dev):
            return pltpu.make_async_remote_copy(
                src_ref=src, dst_ref=dst, send_sem=send, recv_sem=recv,
                device_id=(dev,), device_id_type=MESH)

        send_descs = []


        def p1_stage(h):
            comm_r[h, 0, :, :] = partial(top_rows(p + 3), h).astype(bf16)
            comm_l[h, 0, :, :] = partial(bot_rows(p + 1), h).astype(bf16)

        def p1_start(h, s):
            rr = copy(comm_r.at[h, s], comm_r.at[h, s + 1],
                      r_send.at[h * 3 + s], r_recv.at[h * 3 + s], right)
            ll = copy(comm_l.at[h, s], comm_l.at[h, s + 1],
                      l_send.at[h * 3 + s], l_recv.at[h * 3 + s], left)
            rr.start()
            ll.start()
            return rr, ll

        def p1_finish(h, s, rr, ll):
            p_r = partial(top_rows(p + 2 - s), h)
            p_l = partial(bot_rows(p + 2 + s), h)
            rr.wait()
            ll.wait()
            if s < 2:
                comm_r[h, s + 1, :, :] = (
                    comm_r[h, s + 1, :, :].astype(f32) + p_r).astype(bf16)
                comm_l[h, s + 1, :, :] = (
                    comm_l[h, s + 1, :, :].astype(f32) + p_l).astype(bf16)
            else:
                zred_ref[h, pl.ds(0, half), :] = (
                    comm_r[h, 3, :, :].astype(f32) + p_r).astype(bf16)
                zred_ref[h, pl.ds(half, half), :] = (
                    comm_l[h, 3, :, :].astype(f32) + p_l).astype(bf16)

        def p2_send(h):
            for dz in range(1, 4):
                zq = jnp.mod(zpos + dz, 4)
                d = copy(zred_ref.at[h, pl.ds(zq * chunk, chunk), :],
                         zrs_ref.at[h, pl.ds(zpos * chunk, chunk), :],
                         zrs_send.at[h * 3 + dz - 1],
                         zrs_recv.at[h * 4 + zpos], col_id(zpos + dz))
                d.start()
                send_descs.append(d)

        def p2_reduce_silu(h):
            acc = zred_ref[h, pl.ds(zpos * chunk, chunk), :].astype(f32)
            for dz in range(1, 4):
                zq = jnp.mod(zpos + dz, 4)
                slot = zrs_ref.at[h, pl.ds(zq * chunk, chunk), :]
                copy(slot, slot, zrs_send.at[0],
                     zrs_recv.at[h * 4 + zq], my).wait_recv()
                acc = acc + zrs_ref[h, pl.ds(zq * chunk, chunk), :].astype(f32)
            zs = acc / (1.0 + jnp.exp(-acc))
            out_ref[my_rows, cols(h)] = zs.astype(bf16)

        def p3_send(h):
            for dz in range(1, 4):
                d = copy(out_ref.at[my_rows, cols(h)],
                         out_ref.at[my_rows, cols(h)],
                         zag_send.at[h * 3 + dz - 1],
                         zag_recv.at[h * 4 + zpos], col_id(zpos + dz))
                d.start()
                send_descs.append(d)

        def p3_wait(h):
            for dz in range(1, 4):
                zq = jnp.mod(zpos + dz, 4)
                slot = out_ref.at[pl.ds(p * grp + zq * chunk, chunk), cols(h)]
                copy(slot, slot, zag_send.at[0],
                     zag_recv.at[h * 4 + zq], my).wait_recv()

        def p4_start(h, s):
            sr = copy(out_ref.at[top_rows(p - s), cols(h)],
                      out_ref.at[top_rows(p - s), cols(h)],
                      ag_send.at[h * 6 + s], ag_recv.at[h * 6 + s], right)
            sl = copy(out_ref.at[bot_rows(p + s), cols(h)],
                      out_ref.at[bot_rows(p + s), cols(h)],
                      ag_send.at[h * 6 + 3 + s], ag_recv.at[h * 6 + 3 + s],
                      left)
            sr.start()
            sl.start()
            send_descs.append(sr)
            send_descs.append(sl)

        def p4_finish(h, s):
            slot_r = out_ref.at[top_rows(p - s - 1), cols(h)]
            copy(slot_r, slot_r, ag_send.at[0],
                 ag_recv.at[h * 6 + s], my).wait_recv()
            slot_l = out_ref.at[bot_rows(p + s + 1), cols(h)]
            copy(slot_l, slot_l, ag_send.at[0],
                 ag_recv.at[h * 6 + 3 + s], my).wait_recv()

        partners = (left, right, pr_id(p + 2),
                    col_id(zpos + 1), col_id(zpos + 2), col_id(zpos + 3))
        barrier = pltpu.get_barrier_semaphore()
        for tgt in partners:
            pl.semaphore_signal(barrier, inc=1, device_id=(tgt,),
                                device_id_type=MESH)
        pl.semaphore_wait(barrier, 6)

        d = {}
        p1_stage(0)
        d[0] = p1_start(0, 0)
        p1_stage(1)
        p1_finish(0, 0, *d[0])
        d[0] = p1_start(0, 1)
        d[1] = p1_start(1, 0)
        p1_finish(0, 1, *d[0])
        d[0] = p1_start(0, 2)
        p1_stage(2)
        p1_finish(1, 0, *d[1])
        d[1] = p1_start(1, 1)
        p1_finish(0, 2, *d[0])
        p2_send(0)
        d[2] = p1_start(2, 0)
        p1_finish(1, 1, *d[1])
        d[1] = p1_start(1, 2)
        p2_reduce_silu(0)
        p3_send(0)
        p1_stage(3)
        p1_finish(2, 0, *d[2])
        d[2] = p1_start(2, 1)
        p1_finish(1, 2, *d[1])
        p2_send(1)
        d[3] = p1_start(3, 0)
        p3_wait(0)
        p1_finish(2, 1, *d[2])
        d[2] = p1_start(2, 2)
        p2_reduce_silu(1)
        p3_send(1)
        p4_start(0, 0)
        p1_finish(3, 0, *d[3])
        d[3] = p1_start(3, 1)
        p1_finish(2, 2, *d[2])
        p2_send(2)
        p4_finish(0, 0)
        p4_start(0, 1)
        p3_wait(1)
        p1_finish(3, 1, *d[3])
        d[3] = p1_start(3, 2)
        p2_reduce_silu(2)
        p3_send(2)
        p4_start(1, 0)
        p4_finish(0, 1)
        p4_start(0, 2)
        p1_finish(3, 2, *d[3])
        p2_send(3)
        p4_finish(1, 0)
        p4_start(1, 1)
        p3_wait(2)
        p2_reduce_silu(3)
        p3_send(3)
        p4_start(2, 0)
        p4_finish(0, 2)
        p4_finish(1, 1)
        p4_start(1, 2)
        p3_wait(3)
        p4_finish(2, 0)
        p4_start(2, 1)
        p4_start(3, 0)
        p4_finish(1, 2)
        p4_finish(2, 1)
        p4_start(2, 2)
        p4_finish(3, 0)
        p4_start(3, 1)
        p4_finish(2, 2)
        p4_finish(3, 1)
        p4_start(3, 2)
        p4_finish(3, 2)

        for d in send_descs:
            d.wait_send()

    return pl.pallas_call(
        body,
        out_shape=jax.ShapeDtypeStruct((m, n), bf16),
        in_specs=[
            pl.BlockSpec(memory_space=pltpu.VMEM),
            pl.BlockSpec(memory_space=pltpu.VMEM),
        ],
        out_specs=pl.BlockSpec(memory_space=pltpu.VMEM),
        scratch_shapes=[
            pltpu.VMEM((4, 4, half, nh), bf16),
            pltpu.VMEM((4, 4, half, nh), bf16),
            pltpu.VMEM((4, grp, nh), bf16),
            pltpu.VMEM((4, grp, nh), bf16),
            pltpu.SemaphoreType.DMA((12,)),
            pltpu.SemaphoreType.DMA((12,)),
            pltpu.SemaphoreType.DMA((12,)),
            pltpu.SemaphoreType.DMA((12,)),
            pltpu.SemaphoreType.DMA((12,)),
            pltpu.SemaphoreType.DMA((16,)),
            pltpu.SemaphoreType.DMA((12,)),
            pltpu.SemaphoreType.DMA((16,)),
            pltpu.SemaphoreType.DMA((24,)),
            pltpu.SemaphoreType.DMA((24,)),
        ],
        compiler_params=pltpu.CompilerParams(
            collective_id=0,
            vmem_limit_bytes=45 * 1024 * 1024,
        ),
    )(A, B)
